# baseline (speedup 1.0000x reference)
import functools

import jax
import jax.numpy as jnp
from jax.experimental import pallas as pl
from jax.experimental.pallas import tpu as pltpu

_TM = 1024
_TK = 4096
_TM_XW = 2048
_NODES_PER_GRAPH = 128
_DBC = 8
_VMEM_BUDGET = 48 * 1024 * 1024


def _xw_kernel(x_ref, w_ref, o_ref):
    o_ref[...] = jnp.dot(x_ref[...], w_ref[...],
                         preferred_element_type=jnp.float32).astype(o_ref.dtype)


def _feature_transform(x, w):
    n, f_in = x.shape
    f_out = w.shape[1]
    flops = 2 * n * f_in * f_out
    return pl.pallas_call(
        _xw_kernel,
        out_shape=jax.ShapeDtypeStruct((n, f_out), jnp.bfloat16),
        grid_spec=pltpu.PrefetchScalarGridSpec(
            num_scalar_prefetch=0,
            grid=(n // _TM_XW,),
            in_specs=[
                pl.BlockSpec((_TM_XW, f_in), lambda i: (i, 0)),
                pl.BlockSpec((f_in, f_out), lambda i: (0, 0)),
            ],
            out_specs=pl.BlockSpec((_TM_XW, f_out), lambda i: (i, 0)),
        ),
        compiler_params=pltpu.CompilerParams(
            dimension_semantics=("parallel",),
            vmem_limit_bytes=32 * 1024 * 1024,
        ),
        cost_estimate=pl.CostEstimate(
            flops=flops, transcendentals=0,
            bytes_accessed=int(x.size * 2 + w.size * 2 + n * f_out * 2)),
    )(x, w)


def _agg_kernel(a_ref, h_ref, d_ref, sd_ref, b_ref, o_ref, acc_ref, *, tk,
                tm):
    k = pl.program_id(1)

    @pl.when(k == 0)
    def _():
        acc_ref[...] = jnp.zeros_like(acc_ref)

    start = pl.multiple_of(k * tk, tk)
    acc_ref[...] += jnp.dot(a_ref[...], h_ref[pl.ds(start, tk), :],
                            preferred_element_type=jnp.float32)

    @pl.when(k == pl.num_programs(1) - 1)
    def _():
        i = pl.program_id(0)
        istart = pl.multiple_of(i * tm, tm)
        h_i = h_ref[pl.ds(istart, tm), :].astype(jnp.float32)
        dcol = d_ref[:, :1]
        sdcol = sd_ref[:, :1]
        z = dcol * acc_ref[...] + sdcol * h_i + b_ref[...]
        o_ref[...] = (dcol * jnp.maximum(z, 0.0)).astype(o_ref.dtype)


def _gcn_aggregate_relu(a, h, d_bc, sd_bc, b):
    n = a.shape[0]
    f_out = h.shape[1]
    flops = 2 * n * n * f_out
    return pl.pallas_call(
        functools.partial(_agg_kernel, tk=_TK, tm=_TM),
        out_shape=jax.ShapeDtypeStruct((n, f_out), jnp.bfloat16),
        grid_spec=pltpu.PrefetchScalarGridSpec(
            num_scalar_prefetch=0,
            grid=(n // _TM, n // _TK),
            in_specs=[
                pl.BlockSpec((_TM, _TK), lambda i, k: (i, k)),
                pl.BlockSpec((n, f_out), lambda i, k: (0, 0)),
                pl.BlockSpec((_TM, _DBC), lambda i, k: (i, 0)),
                pl.BlockSpec((_TM, _DBC), lambda i, k: (i, 0)),
                pl.BlockSpec((1, f_out), lambda i, k: (0, 0)),
            ],
            out_specs=pl.BlockSpec((_TM, f_out), lambda i, k: (i, 0)),
            scratch_shapes=[pltpu.VMEM((_TM, f_out), jnp.float32)],
        ),
        compiler_params=pltpu.CompilerParams(
            dimension_semantics=("parallel", "arbitrary"),
            vmem_limit_bytes=_VMEM_BUDGET,
        ),
        cost_estimate=pl.CostEstimate(
            flops=flops, transcendentals=0,
            bytes_accessed=int(a.size * 2 + h.size * 2 + n * f_out * 2)),
    )(a, h, d_bc, sd_bc, b)


def _agg_pool_kernel(a_ref, h_ref, d_ref, sd_ref, b_ref, lw_ref, lb_ref,
                     o_ref, acc_ref, *, tk, tm, graphs_per_tile):
    k = pl.program_id(1)

    @pl.when(k == 0)
    def _():
        acc_ref[...] = jnp.zeros_like(acc_ref)

    start = pl.multiple_of(k * tk, tk)
    acc_ref[...] += jnp.dot(a_ref[...], h_ref[pl.ds(start, tk), :],
                            preferred_element_type=jnp.float32)

    @pl.when(k == pl.num_programs(1) - 1)
    def _():
        i = pl.program_id(0)
        istart = pl.multiple_of(i * tm, tm)
        h_i = h_ref[pl.ds(istart, tm), :].astype(jnp.float32)
        dcol = d_ref[:, :1]
        sdcol = sd_ref[:, :1]
        z = dcol * acc_ref[...] + sdcol * h_i + b_ref[...]
        z = z.astype(jnp.bfloat16).astype(jnp.float32)
        hid = z.shape[-1]
        pooled = jnp.mean(
            z.reshape(graphs_per_tile, _NODES_PER_GRAPH, hid), axis=1)
        logits = jnp.dot(pooled.astype(jnp.bfloat16), lw_ref[...],
                         preferred_element_type=jnp.float32) + lb_ref[...]
        o_ref[...] = logits[None]


def _gcn_aggregate_pool_linear(a, h, d_bc, sd_bc, b, lw, lb):
    n = a.shape[0]
    f_out = h.shape[1]
    out_f = lw.shape[1]
    gpt = _TM // _NODES_PER_GRAPH
    n_tiles = n // _TM
    flops = 2 * n * n * f_out
    out = pl.pallas_call(
        functools.partial(_agg_pool_kernel, tk=_TK, tm=_TM,
                          graphs_per_tile=gpt),
        out_shape=jax.ShapeDtypeStruct((n_tiles, gpt, out_f), jnp.float32),
        grid_spec=pltpu.PrefetchScalarGridSpec(
            num_scalar_prefetch=0,
            grid=(n_tiles, n // _TK),
            in_specs=[
                pl.BlockSpec((_TM, _TK), lambda i, k: (i, k)),
                pl.BlockSpec((n, f_out), lambda i, k: (0, 0)),
                pl.BlockSpec((_TM, _DBC), lambda i, k: (i, 0)),
                pl.BlockSpec((_TM, _DBC), lambda i, k: (i, 0)),
                pl.BlockSpec((1, f_out), lambda i, k: (0, 0)),
                pl.BlockSpec((f_out, out_f), lambda i, k: (0, 0)),
                pl.BlockSpec((1, out_f), lambda i, k: (0, 0)),
            ],
            out_specs=pl.BlockSpec((1, gpt, out_f), lambda i, k: (i, 0, 0)),
            scratch_shapes=[pltpu.VMEM((_TM, f_out), jnp.float32)],
        ),
        compiler_params=pltpu.CompilerParams(
            dimension_semantics=("parallel", "arbitrary"),
            vmem_limit_bytes=_VMEM_BUDGET,
        ),
        cost_estimate=pl.CostEstimate(
            flops=flops, transcendentals=0,
            bytes_accessed=int(a.size * 2 + h.size * 2 + n_tiles * gpt * out_f * 4)),
    )(a, h, d_bc, sd_bc, b, lw, lb)
    return out.reshape(n_tiles * gpt, out_f)


def kernel(x, edge_index, batch, conv_w_0, conv_w_1, conv_b_0, conv_b_1,
           lin_w, lin_b):
    n = x.shape[0]
    src, dst = edge_index[0], edge_index[1]

    is_self = (src == dst).astype(jnp.int32)
    keys = dst * 2 + is_self
    cnt = jnp.zeros((2 * n,), jnp.float32).at[keys].add(
        1.0, mode="promise_in_bounds")
    in_cnt = cnt[0::2] + cnt[1::2]
    self_cnt = cnt[1::2]
    need_loop = (self_cnt == 0).astype(jnp.float32)
    deg = in_cnt + need_loop
    d = jax.lax.rsqrt(deg)
    sd = need_loop * d

    lin = dst * n + src
    a_flat = jnp.zeros((n * n,), jnp.float16).at[lin].add(
        jnp.float16(1.0), mode="promise_in_bounds")
    a = a_flat.astype(jnp.bfloat16).reshape(n, n)

    d_bc = jnp.broadcast_to(d[:, None], (n, _DBC))
    sd_bc = jnp.broadcast_to(sd[:, None], (n, _DBC))

    w0 = conv_w_0.astype(jnp.bfloat16)
    w1 = conv_w_1.astype(jnp.bfloat16)
    b0 = conv_b_0.reshape(1, -1)
    b1 = conv_b_1.reshape(1, -1)
    lw = lin_w.astype(jnp.bfloat16)
    lb = lin_b.reshape(1, -1)

    h = (x * d[:, None]).astype(jnp.bfloat16)
    h = _feature_transform(h, w0)
    h = _gcn_aggregate_relu(a, h, d_bc, sd_bc, b0)
    h = _feature_transform(h, w1)
    logits = _gcn_aggregate_pool_linear(a, h, d_bc, sd_bc, b1, lw, lb)
    return {"logits": logits}


# speedup vs baseline: 1.4985x; 1.4985x over previous
import functools

import jax
import jax.numpy as jnp
from jax.experimental import pallas as pl
from jax.experimental.pallas import tpu as pltpu

_TM = 1024
_TK = 4096
_TM_XW = 2048
_NODES_PER_GRAPH = 128
_DBC = 8
_VMEM_BUDGET = 48 * 1024 * 1024


def _xw_kernel(x_ref, w_ref, o_ref):
    o_ref[...] = jnp.dot(x_ref[...], w_ref[...],
                         preferred_element_type=jnp.float32).astype(o_ref.dtype)


def _feature_transform(x, w):
    n, f_in = x.shape
    f_out = w.shape[1]
    flops = 2 * n * f_in * f_out
    return pl.pallas_call(
        _xw_kernel,
        out_shape=jax.ShapeDtypeStruct((n, f_out), jnp.bfloat16),
        grid_spec=pltpu.PrefetchScalarGridSpec(
            num_scalar_prefetch=0,
            grid=(n // _TM_XW,),
            in_specs=[
                pl.BlockSpec((_TM_XW, f_in), lambda i: (i, 0)),
                pl.BlockSpec((f_in, f_out), lambda i: (0, 0)),
            ],
            out_specs=pl.BlockSpec((_TM_XW, f_out), lambda i: (i, 0)),
        ),
        compiler_params=pltpu.CompilerParams(
            dimension_semantics=("parallel",),
            vmem_limit_bytes=32 * 1024 * 1024,
        ),
        cost_estimate=pl.CostEstimate(
            flops=flops, transcendentals=0,
            bytes_accessed=int(x.size * 2 + w.size * 2 + n * f_out * 2)),
    )(x, w)


def _agg_kernel(a_ref, h_ref, d_ref, sd_ref, b_ref, o_ref, acc_ref, *, tk,
                tm):
    k = pl.program_id(1)

    @pl.when(k == 0)
    def _():
        acc_ref[...] = jnp.zeros_like(acc_ref)

    start = pl.multiple_of(k * tk, tk)
    acc_ref[...] += jnp.dot(a_ref[...], h_ref[pl.ds(start, tk), :],
                            preferred_element_type=jnp.float32)

    @pl.when(k == pl.num_programs(1) - 1)
    def _():
        i = pl.program_id(0)
        istart = pl.multiple_of(i * tm, tm)
        h_i = h_ref[pl.ds(istart, tm), :].astype(jnp.float32)
        dcol = d_ref[:, :1]
        sdcol = sd_ref[:, :1]
        z = dcol * acc_ref[...] + sdcol * h_i + b_ref[...]
        o_ref[...] = (dcol * jnp.maximum(z, 0.0)).astype(o_ref.dtype)


def _gcn_aggregate_relu(a, h, d_bc, sd_bc, b):
    n = a.shape[0]
    f_out = h.shape[1]
    flops = 2 * n * n * f_out
    return pl.pallas_call(
        functools.partial(_agg_kernel, tk=_TK, tm=_TM),
        out_shape=jax.ShapeDtypeStruct((n, f_out), jnp.bfloat16),
        grid_spec=pltpu.PrefetchScalarGridSpec(
            num_scalar_prefetch=0,
            grid=(n // _TM, n // _TK),
            in_specs=[
                pl.BlockSpec((_TM, _TK), lambda i, k: (i, k)),
                pl.BlockSpec((n, f_out), lambda i, k: (0, 0)),
                pl.BlockSpec((_TM, _DBC), lambda i, k: (i, 0)),
                pl.BlockSpec((_TM, _DBC), lambda i, k: (i, 0)),
                pl.BlockSpec((1, f_out), lambda i, k: (0, 0)),
            ],
            out_specs=pl.BlockSpec((_TM, f_out), lambda i, k: (i, 0)),
            scratch_shapes=[pltpu.VMEM((_TM, f_out), jnp.float32)],
        ),
        compiler_params=pltpu.CompilerParams(
            dimension_semantics=("parallel", "arbitrary"),
            vmem_limit_bytes=_VMEM_BUDGET,
        ),
        cost_estimate=pl.CostEstimate(
            flops=flops, transcendentals=0,
            bytes_accessed=int(a.size * 2 + h.size * 2 + n * f_out * 2)),
    )(a, h, d_bc, sd_bc, b)


def _agg_pool_kernel(a_ref, h_ref, d_ref, sd_ref, b_ref, lw_ref, lb_ref,
                     o_ref, acc_ref, *, tk, tm, graphs_per_tile):
    k = pl.program_id(1)

    @pl.when(k == 0)
    def _():
        acc_ref[...] = jnp.zeros_like(acc_ref)

    start = pl.multiple_of(k * tk, tk)
    acc_ref[...] += jnp.dot(a_ref[...], h_ref[pl.ds(start, tk), :],
                            preferred_element_type=jnp.float32)

    @pl.when(k == pl.num_programs(1) - 1)
    def _():
        i = pl.program_id(0)
        istart = pl.multiple_of(i * tm, tm)
        h_i = h_ref[pl.ds(istart, tm), :].astype(jnp.float32)
        dcol = d_ref[:, :1]
        sdcol = sd_ref[:, :1]
        z = dcol * acc_ref[...] + sdcol * h_i + b_ref[...]
        z = z.astype(jnp.bfloat16).astype(jnp.float32)
        hid = z.shape[-1]
        pooled = jnp.mean(
            z.reshape(graphs_per_tile, _NODES_PER_GRAPH, hid), axis=1)
        logits = jnp.dot(pooled.astype(jnp.bfloat16), lw_ref[...],
                         preferred_element_type=jnp.float32) + lb_ref[...]
        o_ref[...] = logits[None]


def _gcn_aggregate_pool_linear(a, h, d_bc, sd_bc, b, lw, lb):
    n = a.shape[0]
    f_out = h.shape[1]
    out_f = lw.shape[1]
    gpt = _TM // _NODES_PER_GRAPH
    n_tiles = n // _TM
    flops = 2 * n * n * f_out
    out = pl.pallas_call(
        functools.partial(_agg_pool_kernel, tk=_TK, tm=_TM,
                          graphs_per_tile=gpt),
        out_shape=jax.ShapeDtypeStruct((n_tiles, gpt, out_f), jnp.float32),
        grid_spec=pltpu.PrefetchScalarGridSpec(
            num_scalar_prefetch=0,
            grid=(n_tiles, n // _TK),
            in_specs=[
                pl.BlockSpec((_TM, _TK), lambda i, k: (i, k)),
                pl.BlockSpec((n, f_out), lambda i, k: (0, 0)),
                pl.BlockSpec((_TM, _DBC), lambda i, k: (i, 0)),
                pl.BlockSpec((_TM, _DBC), lambda i, k: (i, 0)),
                pl.BlockSpec((1, f_out), lambda i, k: (0, 0)),
                pl.BlockSpec((f_out, out_f), lambda i, k: (0, 0)),
                pl.BlockSpec((1, out_f), lambda i, k: (0, 0)),
            ],
            out_specs=pl.BlockSpec((1, gpt, out_f), lambda i, k: (i, 0, 0)),
            scratch_shapes=[pltpu.VMEM((_TM, f_out), jnp.float32)],
        ),
        compiler_params=pltpu.CompilerParams(
            dimension_semantics=("parallel", "arbitrary"),
            vmem_limit_bytes=_VMEM_BUDGET,
        ),
        cost_estimate=pl.CostEstimate(
            flops=flops, transcendentals=0,
            bytes_accessed=int(a.size * 2 + h.size * 2 + n_tiles * gpt * out_f * 4)),
    )(a, h, d_bc, sd_bc, b, lw, lb)
    return out.reshape(n_tiles * gpt, out_f)


def kernel(x, edge_index, batch, conv_w_0, conv_w_1, conv_b_0, conv_b_1,
           lin_w, lin_b):
    n = x.shape[0]
    src, dst = edge_index[0], edge_index[1]

    is_self = (src == dst).astype(jnp.int32)
    keys = n * n + dst * 2 + is_self
    lin = dst * n + src
    flat = jnp.zeros((n * n + 2 * n,), jnp.float32).at[
        jnp.concatenate([lin, keys])].add(1.0, mode="promise_in_bounds")
    cnt = flat[n * n:]
    in_cnt = cnt[0::2] + cnt[1::2]
    self_cnt = cnt[1::2]
    need_loop = (self_cnt == 0).astype(jnp.float32)
    deg = in_cnt + need_loop
    d = jax.lax.rsqrt(deg)
    sd = need_loop * d

    a = flat[:n * n].astype(jnp.bfloat16).reshape(n, n)

    d_bc = jnp.broadcast_to(d[:, None], (n, _DBC))
    sd_bc = jnp.broadcast_to(sd[:, None], (n, _DBC))

    w0 = conv_w_0.astype(jnp.bfloat16)
    w1 = conv_w_1.astype(jnp.bfloat16)
    b0 = conv_b_0.reshape(1, -1)
    b1 = conv_b_1.reshape(1, -1)
    lw = lin_w.astype(jnp.bfloat16)
    lb = lin_b.reshape(1, -1)

    h = (x * d[:, None]).astype(jnp.bfloat16)
    h = _feature_transform(h, w0)
    h = _gcn_aggregate_relu(a, h, d_bc, sd_bc, b0)
    h = _feature_transform(h, w1)
    logits = _gcn_aggregate_pool_linear(a, h, d_bc, sd_bc, b1, lw, lb)
    return {"logits": logits}


# speedup vs baseline: 1.5252x; 1.0178x over previous
import functools

import jax
import jax.numpy as jnp
from jax.experimental import pallas as pl
from jax.experimental.pallas import tpu as pltpu

_TM = 1024
_TK = 4096
_TM_XW = 2048
_NODES_PER_GRAPH = 128
_DBC = 8
_VMEM_BUDGET = 48 * 1024 * 1024


def _xw_kernel(x_ref, w_ref, o_ref):
    o_ref[...] = jnp.dot(x_ref[...], w_ref[...],
                         preferred_element_type=jnp.float32).astype(o_ref.dtype)


def _feature_transform(x, w):
    n, f_in = x.shape
    f_out = w.shape[1]
    flops = 2 * n * f_in * f_out
    return pl.pallas_call(
        _xw_kernel,
        out_shape=jax.ShapeDtypeStruct((n, f_out), jnp.bfloat16),
        grid_spec=pltpu.PrefetchScalarGridSpec(
            num_scalar_prefetch=0,
            grid=(n // _TM_XW,),
            in_specs=[
                pl.BlockSpec((_TM_XW, f_in), lambda i: (i, 0)),
                pl.BlockSpec((f_in, f_out), lambda i: (0, 0)),
            ],
            out_specs=pl.BlockSpec((_TM_XW, f_out), lambda i: (i, 0)),
        ),
        compiler_params=pltpu.CompilerParams(
            dimension_semantics=("parallel",),
            vmem_limit_bytes=32 * 1024 * 1024,
        ),
        cost_estimate=pl.CostEstimate(
            flops=flops, transcendentals=0,
            bytes_accessed=int(x.size * 2 + w.size * 2 + n * f_out * 2)),
    )(x, w)


def _agg_kernel(a_ref, h_ref, d_ref, sd_ref, b_ref, o_ref, acc_ref, *, tk,
                tm):
    k = pl.program_id(1)

    @pl.when(k == 0)
    def _():
        acc_ref[...] = jnp.zeros_like(acc_ref)

    start = pl.multiple_of(k * tk, tk)
    acc_ref[...] += jnp.dot(a_ref[...], h_ref[pl.ds(start, tk), :],
                            preferred_element_type=jnp.float32)

    @pl.when(k == pl.num_programs(1) - 1)
    def _():
        i = pl.program_id(0)
        istart = pl.multiple_of(i * tm, tm)
        h_i = h_ref[pl.ds(istart, tm), :].astype(jnp.float32)
        dcol = d_ref[:, :1]
        sdcol = sd_ref[:, :1]
        z = dcol * acc_ref[...] + sdcol * h_i + b_ref[...]
        o_ref[...] = (dcol * jnp.maximum(z, 0.0)).astype(o_ref.dtype)


def _gcn_aggregate_relu(a, h, d_bc, sd_bc, b):
    n = a.shape[0]
    f_out = h.shape[1]
    flops = 2 * n * n * f_out
    return pl.pallas_call(
        functools.partial(_agg_kernel, tk=_TK, tm=_TM),
        out_shape=jax.ShapeDtypeStruct((n, f_out), jnp.bfloat16),
        grid_spec=pltpu.PrefetchScalarGridSpec(
            num_scalar_prefetch=0,
            grid=(n // _TM, n // _TK),
            in_specs=[
                pl.BlockSpec((_TM, _TK), lambda i, k: (i, k)),
                pl.BlockSpec((n, f_out), lambda i, k: (0, 0)),
                pl.BlockSpec((_TM, _DBC), lambda i, k: (i, 0)),
                pl.BlockSpec((_TM, _DBC), lambda i, k: (i, 0)),
                pl.BlockSpec((1, f_out), lambda i, k: (0, 0)),
            ],
            out_specs=pl.BlockSpec((_TM, f_out), lambda i, k: (i, 0)),
            scratch_shapes=[pltpu.VMEM((_TM, f_out), jnp.float32)],
        ),
        compiler_params=pltpu.CompilerParams(
            dimension_semantics=("parallel", "arbitrary"),
            vmem_limit_bytes=_VMEM_BUDGET,
        ),
        cost_estimate=pl.CostEstimate(
            flops=flops, transcendentals=0,
            bytes_accessed=int(a.size * 2 + h.size * 2 + n * f_out * 2)),
    )(a, h, d_bc, sd_bc, b)


def _agg_pool_kernel(a_ref, h_ref, d_ref, sd_ref, b_ref, lw_ref, lb_ref,
                     o_ref, acc_ref, *, tk, tm, graphs_per_tile):
    k = pl.program_id(1)

    @pl.when(k == 0)
    def _():
        acc_ref[...] = jnp.zeros_like(acc_ref)

    start = pl.multiple_of(k * tk, tk)
    acc_ref[...] += jnp.dot(a_ref[...], h_ref[pl.ds(start, tk), :],
                            preferred_element_type=jnp.float32)

    @pl.when(k == pl.num_programs(1) - 1)
    def _():
        i = pl.program_id(0)
        istart = pl.multiple_of(i * tm, tm)
        h_i = h_ref[pl.ds(istart, tm), :].astype(jnp.float32)
        dcol = d_ref[:, :1]
        sdcol = sd_ref[:, :1]
        z = dcol * acc_ref[...] + sdcol * h_i + b_ref[...]
        z = z.astype(jnp.bfloat16).astype(jnp.float32)
        hid = z.shape[-1]
        pooled = jnp.mean(
            z.reshape(graphs_per_tile, _NODES_PER_GRAPH, hid), axis=1)
        logits = jnp.dot(pooled.astype(jnp.bfloat16), lw_ref[...],
                         preferred_element_type=jnp.float32) + lb_ref[...]
        o_ref[...] = logits[None]


def _gcn_aggregate_pool_linear(a, h, d_bc, sd_bc, b, lw, lb):
    n = a.shape[0]
    f_out = h.shape[1]
    out_f = lw.shape[1]
    gpt = _TM // _NODES_PER_GRAPH
    n_tiles = n // _TM
    flops = 2 * n * n * f_out
    out = pl.pallas_call(
        functools.partial(_agg_pool_kernel, tk=_TK, tm=_TM,
                          graphs_per_tile=gpt),
        out_shape=jax.ShapeDtypeStruct((n_tiles, gpt, out_f), jnp.float32),
        grid_spec=pltpu.PrefetchScalarGridSpec(
            num_scalar_prefetch=0,
            grid=(n_tiles, n // _TK),
            in_specs=[
                pl.BlockSpec((_TM, _TK), lambda i, k: (i, k)),
                pl.BlockSpec((n, f_out), lambda i, k: (0, 0)),
                pl.BlockSpec((_TM, _DBC), lambda i, k: (i, 0)),
                pl.BlockSpec((_TM, _DBC), lambda i, k: (i, 0)),
                pl.BlockSpec((1, f_out), lambda i, k: (0, 0)),
                pl.BlockSpec((f_out, out_f), lambda i, k: (0, 0)),
                pl.BlockSpec((1, out_f), lambda i, k: (0, 0)),
            ],
            out_specs=pl.BlockSpec((1, gpt, out_f), lambda i, k: (i, 0, 0)),
            scratch_shapes=[pltpu.VMEM((_TM, f_out), jnp.float32)],
        ),
        compiler_params=pltpu.CompilerParams(
            dimension_semantics=("parallel", "arbitrary"),
            vmem_limit_bytes=_VMEM_BUDGET,
        ),
        cost_estimate=pl.CostEstimate(
            flops=flops, transcendentals=0,
            bytes_accessed=int(a.size * 2 + h.size * 2 + n_tiles * gpt * out_f * 4)),
    )(a, h, d_bc, sd_bc, b, lw, lb)
    return out.reshape(n_tiles * gpt, out_f)


def kernel(x, edge_index, batch, conv_w_0, conv_w_1, conv_b_0, conv_b_1,
           lin_w, lin_b):
    n = x.shape[0]
    src, dst = edge_index[0], edge_index[1]

    is_self = (src == dst).astype(jnp.int32)
    keys = dst * 2 + is_self
    cnt = jnp.zeros((2 * n,), jnp.float32).at[keys].add(
        1.0, mode="promise_in_bounds")
    in_cnt = cnt[0::2] + cnt[1::2]
    self_cnt = cnt[1::2]
    need_loop = (self_cnt == 0).astype(jnp.float32)
    deg = in_cnt + need_loop
    d = jax.lax.rsqrt(deg)
    sd = need_loop * d

    lin = dst * n + src
    a_flat = jnp.zeros((n * n,), jnp.float32).at[lin].add(
        1.0, mode="promise_in_bounds")
    a = a_flat.astype(jnp.bfloat16).reshape(n, n)

    d_bc = jnp.broadcast_to(d[:, None], (n, _DBC))
    sd_bc = jnp.broadcast_to(sd[:, None], (n, _DBC))

    w0 = conv_w_0.astype(jnp.bfloat16)
    w1 = conv_w_1.astype(jnp.bfloat16)
    b0 = conv_b_0.reshape(1, -1)
    b1 = conv_b_1.reshape(1, -1)
    lw = lin_w.astype(jnp.bfloat16)
    lb = lin_b.reshape(1, -1)

    h = (x * d[:, None]).astype(jnp.bfloat16)
    h = _feature_transform(h, w0)
    h = _gcn_aggregate_relu(a, h, d_bc, sd_bc, b0)
    h = _feature_transform(h, w1)
    logits = _gcn_aggregate_pool_linear(a, h, d_bc, sd_bc, b1, lw, lb)
    return {"logits": logits}
